# baseline (speedup 1.0000x reference)
import functools

import jax
import jax.numpy as jnp
from jax import lax
from jax.experimental import pallas as pl
from jax.experimental.pallas import tpu as pltpu
from jax.experimental.pallas import tpu_sc as plsc

_V = 1000000
_D = 64
_B = 4096
_L = 200
_NC = 2
_NS = 16
_NW = _NC * _NS
_BW = 128

_mesh = plsc.VectorSubcoreMesh(core_axis_name="c", subcore_axis_name="s")


@functools.partial(
    pl.kernel,
    mesh=_mesh,
    compiler_params=pltpu.CompilerParams(needs_layout_passes=False),
    out_type=jax.ShapeDtypeStruct((_L, 8, _NW, 8, 128), jnp.float32),
    scratch_types=[
        pltpu.VMEM((_L, _BW), jnp.int32),
        [pltpu.VMEM((_BW,), jnp.int32) for _ in range(2)],
        [pltpu.VMEM((_BW, 133), jnp.float32) for _ in range(2)],
        [pltpu.VMEM((_D, 128), jnp.float32) for _ in range(2)],
        [pltpu.SemaphoreType.DMA for _ in range(2)],
        [pltpu.SemaphoreType.DMA for _ in range(2)],
    ],
)
def _embedding_gather(
    w_hbm, idx_hbm, out_hbm,
    idx_v, irows, gbufs, obufs, gsems, osems,
):
    wid = lax.axis_index("s") * _NC + lax.axis_index("c")

    pltpu.sync_copy(idx_hbm.at[:, pl.ds(wid * _BW, _BW)], idx_v)

    def prep_gather(l, p):
        for k in range(8):
            iv = idx_v[l, pl.ds(16 * k, 16)]
            irows[p][pl.ds(16 * k, 16)] = lax.shift_right_logical(iv, 1)

    def gather_cp(p):
        return pltpu.make_async_copy(
            w_hbm.at[irows[p]], gbufs[p].at[:, pl.ds(0, 2 * _D)], gsems[p]
        )

    def out_cp(l, p, di):
        return pltpu.make_async_copy(
            obufs[p].at[pl.ds(di * 8, 8)],
            out_hbm.at[l, di, wid],
            osems[p],
        )

    def out_start(l, p):
        for di in range(8):
            out_cp(l, p, di).start()

    def out_wait(l, p):
        for di in range(8):
            out_cp(l, p, di).wait()

    iota16 = lax.iota(jnp.int32, 16)
    crows = [iota16 + 16 * cg for cg in range(8)]

    def transpose(l, p):
        ch = []
        for cg in range(8):
            iv = idx_v[l, pl.ds(16 * cg, 16)]
            ch.append((iv & 1) * _D)

        @pl.loop(0, _D, unroll=4)
        def _per_d(d):
            db = jnp.full((16,), 0, jnp.int32) + d
            for cg in range(8):
                vals = plsc.load_gather(gbufs[p], [crows[cg], ch[cg] + db])
                obufs[p][d, pl.ds(16 * cg, 16)] = vals

    def start_step(l, p):
        prep_gather(l, p)
        gather_cp(p).start()

    start_step(0, 0)
    start_step(1, 1)
    gather_cp(0).wait()
    transpose(0, 0)
    out_start(0, 0)
    start_step(2, 0)
    gather_cp(1).wait()
    transpose(1, 1)
    out_start(1, 1)
    start_step(3, 1)

    @pl.loop(0, (_L - 4) // 2)
    def _steady(i):
        for p in range(2):
            l = 2 * i + 2 + p
            gather_cp(p).wait()
            out_wait(l - 2, p)
            transpose(l, p)
            out_start(l, p)
            start_step(l + 2, p)

    for p in range(2):
        l = _L - 2 + p
        gather_cp(p).wait()
        out_wait(l - 2, p)
        transpose(l, p)
        out_start(l, p)
    for p in range(2):
        out_wait(_L - 2 + p, p)


def kernel(x, w):
    out5 = _embedding_gather(w.reshape(_V // 2, 2 * _D), x.T)
    return out5.transpose(2, 4, 0, 1, 3).reshape(_B, _L, _D)
